# baseline (speedup 1.0000x reference)
import jax
import jax.numpy as jnp
from jax import lax
from jax.experimental import pallas as pl
from jax.experimental.pallas import tpu as pltpu
from jax.experimental.pallas import tpu_sc as plsc

BATCH = 16384
DIM = 32
NC = 2
NS = 16
NW = NC * NS
B_PER_W = BATCH // NW
GROUPS = B_PER_W // 16


def _gmf_body(uidx_hbm, iidx_hbm, user_table, item_table, w_hbm, b_hbm,
              out_hbm,
              uidx_v, iidx_v, u_rows, i_rows, out_v, w_v, b_v,
              sem_u, sem_i):
    wid = lax.axis_index("s") * NC + lax.axis_index("c")
    base = wid * B_PER_W

    pltpu.sync_copy(uidx_hbm.at[pl.ds(base, B_PER_W)], uidx_v)
    pltpu.sync_copy(iidx_hbm.at[pl.ds(base, B_PER_W)], iidx_v)
    cu = pltpu.async_copy(user_table.at[uidx_v], u_rows, sem_u)
    ci = pltpu.async_copy(item_table.at[iidx_v], i_rows, sem_i)
    pltpu.sync_copy(w_hbm, w_v)
    pltpu.sync_copy(b_hbm, b_v)
    cu.wait()
    ci.wait()

    lanes = lax.iota(jnp.int32, 16)
    w_lo = w_v[pl.ds(0, 16)]
    w_hi = w_v[pl.ds(16, 16)]
    bias = b_v[...]

    def group(g, carry):
        acc = jnp.zeros((16,), jnp.float32)
        for j in range(16):
            r = g * 16 + j
            ua = u_rows[r, pl.ds(0, 16)]
            ub = u_rows[r, pl.ds(16, 16)]
            ia = i_rows[r, pl.ds(0, 16)]
            ib = i_rows[r, pl.ds(16, 16)]
            p = ua * ia * w_lo + ub * ib * w_hi
            s = jnp.sum(p)
            acc = jnp.where(lanes == j, s, acc)
        logit = acc + bias
        out_v[pl.ds(g * 16, 16)] = 1.0 / (1.0 + jnp.exp(-logit))
        return carry

    lax.fori_loop(0, GROUPS, group, 0)
    pltpu.sync_copy(out_v, out_hbm.at[pl.ds(base, B_PER_W)])


@jax.jit
def kernel(user_indices, item_indices, user_table, item_table, W, b):
    uidx = user_indices.astype(jnp.int32)
    iidx = item_indices.astype(jnp.int32)
    w32 = W.reshape(DIM).astype(jnp.float32)
    b16 = jnp.broadcast_to(b.astype(jnp.float32), (16,))

    run = pl.kernel(
        _gmf_body,
        out_type=jax.ShapeDtypeStruct((BATCH,), jnp.float32),
        mesh=plsc.VectorSubcoreMesh(core_axis_name="c", subcore_axis_name="s"),
        compiler_params=pltpu.CompilerParams(
            needs_layout_passes=False, use_tc_tiling_on_sc=False),
        scratch_types=[
            pltpu.VMEM((B_PER_W,), jnp.int32),
            pltpu.VMEM((B_PER_W,), jnp.int32),
            pltpu.VMEM((B_PER_W, DIM), jnp.float32),
            pltpu.VMEM((B_PER_W, DIM), jnp.float32),
            pltpu.VMEM((B_PER_W,), jnp.float32),
            pltpu.VMEM((DIM,), jnp.float32),
            pltpu.VMEM((16,), jnp.float32),
            pltpu.SemaphoreType.DMA,
            pltpu.SemaphoreType.DMA,
        ],
    )
    out = run(uidx, iidx, user_table, item_table, w32, b16)
    return out.reshape(BATCH, 1)
